# baseline (speedup 1.0000x reference)
import functools
import math

import jax
import jax.numpy as jnp
from jax import lax
from jax.experimental import pallas as pl
from jax.experimental.pallas import tpu as pltpu
from jax.experimental.pallas import tpu_sc as plsc

D_MODEL = 64
SCALE = math.sqrt(D_MODEL)

NC = 2
NS = 16
NW = NC * NS

SEQ = 200
BATCH = 4096
IDXW = 128
CHUNK = 512
GPC = CHUNK // IDXW
QPT = BATCH // CHUNK
NBLOCKS = SEQ * QPT
BPW = NBLOCKS // NW
SLAB = CHUNK * D_MODEL // 8


def _emb_kernel(table_hbm, idx_hbm, out_hbm, idx_v, rows_v, stage_v, sem, osem):
    wid = lax.axis_index("s") * NC + lax.axis_index("c")
    lane = lax.broadcasted_iota(jnp.int32, (16,), 0)

    def block_body(bi, _):
        bid = wid * BPW + bi
        t = bid // QPT
        q = lax.rem(bid, QPT)
        pltpu.sync_copy(idx_hbm.at[t, pl.ds(q * CHUNK, CHUNK)], idx_v)
        copies = []
        for j in range(GPC):
            copies.append(
                pltpu.async_copy(
                    table_hbm.at[idx_v.at[pl.ds(j * IDXW, IDXW)]],
                    rows_v.at[pl.ds(j * IDXW, IDXW)],
                    sem,
                )
            )
        for c in copies:
            c.wait()

        def xp_body(i, _):
            d = i >> 5
            g = i & 31
            row0 = g * 16
            vals = plsc.load_gather(rows_v, [row0 + lane, jnp.full((16,), d, jnp.int32)])
            stage_v[d >> 3, g >> 3, d & 7, pl.ds((g & 7) * 16, 16)] = vals * SCALE
            return 0

        lax.fori_loop(0, 2048, xp_body, 0, unroll=8)

        ocopies = []
        for rr in range(8):
            ocopies.append(
                pltpu.async_copy(
                    stage_v.at[rr],
                    out_hbm.at[t, rr, pl.ds(q * GPC, GPC)],
                    osem,
                )
            )
        for c in ocopies:
            c.wait()
        return 0

    lax.fori_loop(0, BPW, block_body, 0)


@jax.jit
def kernel(x, table):
    xt = jnp.swapaxes(x, 0, 1)
    mesh = plsc.VectorSubcoreMesh(core_axis_name="c", subcore_axis_name="s")
    out3 = pl.kernel(
        _emb_kernel,
        out_type=jax.ShapeDtypeStruct((SEQ, 8, BATCH // IDXW, 8, IDXW), jnp.float32),
        mesh=mesh,
        scratch_types=[
            pltpu.VMEM((CHUNK,), jnp.int32),
            pltpu.VMEM((CHUNK, D_MODEL), jnp.float32),
            pltpu.VMEM((8, GPC, 8, IDXW), jnp.float32),
            pltpu.SemaphoreType.DMA,
            pltpu.SemaphoreType.DMA,
        ],
        compiler_params=pltpu.CompilerParams(
            use_tc_tiling_on_sc=False, needs_layout_passes=False
        ),
    )(table, xt)
    return lax.reshape(out3, (BATCH, SEQ, D_MODEL), dimensions=(2, 4, 0, 1, 3))


# speedup vs baseline: 2.6370x; 2.6370x over previous
import functools
import math

import jax
import jax.numpy as jnp
from jax import lax
from jax.experimental import pallas as pl
from jax.experimental.pallas import tpu as pltpu
from jax.experimental.pallas import tpu_sc as plsc

D_MODEL = 64
SCALE = math.sqrt(D_MODEL)

NC = 2
NS = 16
NW = NC * NS

SEQ = 200
BATCH = 4096
IDXW = 128
CHUNK = 512
GPC = CHUNK // IDXW
QPT = BATCH // CHUNK
NBLOCKS = SEQ * QPT
BPW = NBLOCKS // NW
SLAB = CHUNK * D_MODEL // 8


def _emb_kernel(table_hbm, idx_hbm, out_hbm, idx_v, rows_v, stage_v, sem, osem):
    wid = lax.axis_index("s") * NC + lax.axis_index("c")
    lane = lax.broadcasted_iota(jnp.int32, (16,), 0)

    def block_body(bi, _):
        bid = wid * BPW + bi
        t = bid // QPT
        q = lax.rem(bid, QPT)
        pltpu.sync_copy(idx_hbm.at[t, pl.ds(q * CHUNK, CHUNK)], idx_v)
        copies = []
        for j in range(GPC):
            copies.append(
                pltpu.async_copy(
                    table_hbm.at[idx_v.at[pl.ds(j * IDXW, IDXW)]],
                    rows_v.at[pl.ds(j * IDXW, IDXW)],
                    sem,
                )
            )
        for c in copies:
            c.wait()

        @functools.partial(plsc.parallel_loop, 0, 2048, unroll=8)
        def _(i):
            d = i >> 5
            g = i & 31
            row0 = g * 16
            vals = plsc.load_gather(rows_v, [row0 + lane, jnp.full((16,), d, jnp.int32)])
            stage_v[d >> 3, g >> 3, d & 7, pl.ds((g & 7) * 16, 16)] = vals * SCALE

        ocopies = []
        for rr in range(8):
            ocopies.append(
                pltpu.async_copy(
                    stage_v.at[rr],
                    out_hbm.at[t, rr, pl.ds(q * GPC, GPC)],
                    osem,
                )
            )
        for c in ocopies:
            c.wait()
        return 0

    lax.fori_loop(0, BPW, block_body, 0)


@jax.jit
def kernel(x, table):
    xt = jnp.swapaxes(x, 0, 1)
    mesh = plsc.VectorSubcoreMesh(core_axis_name="c", subcore_axis_name="s")
    out3 = pl.kernel(
        _emb_kernel,
        out_type=jax.ShapeDtypeStruct((SEQ, 8, BATCH // IDXW, 8, IDXW), jnp.float32),
        mesh=mesh,
        scratch_types=[
            pltpu.VMEM((CHUNK,), jnp.int32),
            pltpu.VMEM((CHUNK, D_MODEL), jnp.float32),
            pltpu.VMEM((8, GPC, 8, IDXW), jnp.float32),
            pltpu.SemaphoreType.DMA,
            pltpu.SemaphoreType.DMA,
        ],
        compiler_params=pltpu.CompilerParams(
            use_tc_tiling_on_sc=False, needs_layout_passes=False
        ),
    )(table, xt)
    return lax.reshape(out3, (BATCH, SEQ, D_MODEL), dimensions=(2, 4, 0, 1, 3))
